# baseline (device time: 215900 ns/iter reference)
import jax
import jax.numpy as jnp
from jax import lax
from jax.experimental import pallas as pl
from jax.experimental.pallas import tpu as pltpu

M = 16384
NOUT = 1024
HALF = M // 2
NCH = 32
RCH = HALF // NCH
NS = 8
NSX = 4
YAHEAD = 4
LAHEAD = 6


def kernel(x):
    def body(x_hbm, out_hbm,
             a_f32, b_f32, ysend, yrecv, xsend, xrecv,
             a_sems, b_sems, sum_store_sems, xstore_sems,
             ysend_sems, yrecv_sems, xsend_sems, xrecv_sems):
        my_x = lax.axis_index("x")
        my_y = lax.axis_index("y")
        my_z = lax.axis_index("z")
        peer_y = (my_x, 1 - my_y, my_z)
        peer_x = (1 - my_x, my_y, my_z)
        my_col = my_y * NOUT
        peer_col = (1 - my_y) * NOUT
        row0 = my_x * HALF
        prow0 = (1 - my_x) * HALF

        def start_loads(k):
            s = k % NS
            la = pltpu.make_async_copy(
                x_hbm.at[0, pl.ds(row0 + k * RCH, RCH), pl.ds(peer_col, NOUT)],
                a_f32.at[s], a_sems.at[s])
            lb = pltpu.make_async_copy(
                x_hbm.at[0, pl.ds(row0 + k * RCH, RCH), pl.ds(my_col, NOUT)],
                b_f32.at[s], b_sems.at[s])
            la.start()
            lb.start()
            return la, lb

        def y_rdma(k):
            return pltpu.make_async_remote_copy(
                src_ref=ysend.at[k % NS],
                dst_ref=yrecv.at[pl.ds(k * RCH, RCH), :],
                send_sem=ysend_sems.at[k % NS],
                recv_sem=yrecv_sems.at[k],
                device_id=peer_y, device_id_type=pl.DeviceIdType.MESH)

        def x_rdma(k):
            return pltpu.make_async_remote_copy(
                src_ref=xsend.at[k % NSX],
                dst_ref=xrecv.at[pl.ds(k * RCH, RCH), :],
                send_sem=xsend_sems.at[k % NSX],
                recv_sem=xrecv_sems.at[k],
                device_id=peer_x, device_id_type=pl.DeviceIdType.MESH)

        def issue_ysend(k):
            loads[k][0].wait()
            if k >= NS:
                ys[k - NS].wait_send()
                ys_waited.add(k - NS)
            ysend[k % NS] = a_f32[k % NS].astype(jnp.bfloat16)
            ys[k] = y_rdma(k)
            ys[k].start()

        def drain_xrecv(j):
            x_rdma(j).wait_recv()
            if j >= NSX:
                xstores[j - NSX].wait()
            xst = pltpu.make_async_copy(
                xrecv.at[pl.ds(j * RCH, RCH), :],
                out_hbm.at[pl.ds(prow0 + j * RCH, RCH), :],
                xstore_sems.at[j % NSX])
            xst.start()
            xstores[j] = xst

        loads, ys, xs, sum_stores, xstores = {}, {}, {}, {}, {}
        ys_waited = set()
        for k in range(LAHEAD):
            loads[k] = start_loads(k)

        barrier = pltpu.get_barrier_semaphore()
        for nbr in (peer_y, peer_x):
            pl.semaphore_signal(barrier, inc=1, device_id=nbr,
                                device_id_type=pl.DeviceIdType.MESH)
        pl.semaphore_wait(barrier, 2)

        for k in range(YAHEAD):
            issue_ysend(k)

        for k in range(NCH):
            s = k % NSX
            if k + LAHEAD < NCH:
                loads[k + LAHEAD] = start_loads(k + LAHEAD)
            if k + YAHEAD < NCH:
                issue_ysend(k + YAHEAD)

            ys[k].wait_recv()
            loads[k][1].wait()
            if k >= NSX:
                sum_stores[k - NSX].wait()
                xs[k - NSX].wait_send()
            xsend[s] = (
                b_f32[k % NS] + yrecv[pl.ds(k * RCH, RCH), :].astype(
                    jnp.float32)
            ).astype(jnp.bfloat16)

            st = pltpu.make_async_copy(
                xsend.at[s], out_hbm.at[pl.ds(row0 + k * RCH, RCH), :],
                sum_store_sems.at[s])
            st.start()
            sum_stores[k] = st
            xs[k] = x_rdma(k)
            xs[k].start()

            if k >= 1:
                drain_xrecv(k - 1)

        drain_xrecv(NCH - 1)
        for k in range(NCH - NSX, NCH):
            sum_stores[k].wait()
            xs[k].wait_send()
        for k in range(NCH):
            if k not in ys_waited:
                ys[k].wait_send()
        for k in range(NCH - NSX, NCH):
            xstores[k].wait()

    return pl.pallas_call(
        body,
        out_shape=jax.ShapeDtypeStruct((M, NOUT), jnp.bfloat16),
        in_specs=[pl.BlockSpec(memory_space=pl.ANY)],
        out_specs=pl.BlockSpec(memory_space=pl.ANY),
        scratch_shapes=[
            pltpu.VMEM((NS, RCH, NOUT), jnp.float32),
            pltpu.VMEM((NS, RCH, NOUT), jnp.float32),
            pltpu.VMEM((NS, RCH, NOUT), jnp.bfloat16),
            pltpu.VMEM((HALF, NOUT), jnp.bfloat16),
            pltpu.VMEM((NSX, RCH, NOUT), jnp.bfloat16),
            pltpu.VMEM((HALF, NOUT), jnp.bfloat16),
            pltpu.SemaphoreType.DMA((NS,)),
            pltpu.SemaphoreType.DMA((NS,)),
            pltpu.SemaphoreType.DMA((NSX,)),
            pltpu.SemaphoreType.DMA((NSX,)),
            pltpu.SemaphoreType.DMA((NS,)),
            pltpu.SemaphoreType.DMA((NCH,)),
            pltpu.SemaphoreType.DMA((NSX,)),
            pltpu.SemaphoreType.DMA((NCH,)),
        ],
        compiler_params=pltpu.CompilerParams(
            collective_id=0, vmem_limit_bytes=100 * 1024 * 1024),
    )(x)


# device time: 215550 ns/iter; 1.0016x vs baseline; 1.0016x over previous
import jax
import jax.numpy as jnp
from jax import lax
from jax.experimental import pallas as pl
from jax.experimental.pallas import tpu as pltpu

M = 16384
NOUT = 1024
HALF = M // 2
NCH = 32
RCH = HALF // NCH
NS = 8
NSX = 4
YAHEAD = 4
LAHEAD = 6


def kernel(x):
    def body(x_hbm, out_hbm,
             a_f32, b_f32, ysend, yrecv, xsend, xrecv,
             a_sems, b_sems, sum_store_sems, xstore_sems,
             ysend_sems, yrecv_sems, xsend_sems, xrecv_sems):
        my_x = lax.axis_index("x")
        my_y = lax.axis_index("y")
        my_z = lax.axis_index("z")
        peer_y = (my_x, 1 - my_y, my_z)
        peer_x = (1 - my_x, my_y, my_z)
        my_col = my_y * NOUT
        peer_col = (1 - my_y) * NOUT
        row0 = my_x * HALF
        prow0 = (1 - my_x) * HALF

        def start_loads(k):
            s = k % NS
            la = pltpu.make_async_copy(
                x_hbm.at[0, pl.ds(row0 + k * RCH, RCH), pl.ds(peer_col, NOUT)],
                a_f32.at[s], a_sems.at[s])
            lb = pltpu.make_async_copy(
                x_hbm.at[0, pl.ds(row0 + k * RCH, RCH), pl.ds(my_col, NOUT)],
                b_f32.at[s], b_sems.at[s])
            la.start()
            lb.start()
            return la, lb

        def y_rdma(k):
            return pltpu.make_async_remote_copy(
                src_ref=ysend.at[k % NS],
                dst_ref=yrecv.at[pl.ds(k * RCH, RCH), :],
                send_sem=ysend_sems.at[k % NS],
                recv_sem=yrecv_sems.at[k],
                device_id=peer_y, device_id_type=pl.DeviceIdType.MESH)

        def x_rdma(k):
            return pltpu.make_async_remote_copy(
                src_ref=xsend.at[k % NSX],
                dst_ref=xrecv.at[pl.ds(k * RCH, RCH), :],
                send_sem=xsend_sems.at[k % NSX],
                recv_sem=xrecv_sems.at[k],
                device_id=peer_x, device_id_type=pl.DeviceIdType.MESH)

        def issue_ysend(k):
            loads[k][0].wait()
            if k >= NS:
                ys[k - NS].wait_send()
                ys_waited.add(k - NS)
            ysend[k % NS] = a_f32[k % NS].astype(jnp.bfloat16)
            ys[k] = y_rdma(k)
            ys[k].start()

        def drain_xrecv(j):
            x_rdma(j).wait_recv()
            if j >= NSX:
                xstores[j - NSX].wait()
            xst = pltpu.make_async_copy(
                xrecv.at[pl.ds(j * RCH, RCH), :],
                out_hbm.at[pl.ds(prow0 + j * RCH, RCH), :],
                xstore_sems.at[j % NSX])
            xst.start()
            xstores[j] = xst

        loads, ys, xs, sum_stores, xstores = {}, {}, {}, {}, {}
        ys_waited = set()
        for k in range(LAHEAD):
            loads[k] = start_loads(k)

        barrier = pltpu.get_barrier_semaphore()
        for nbr in (peer_y, peer_x):
            pl.semaphore_signal(barrier, inc=1, device_id=nbr,
                                device_id_type=pl.DeviceIdType.MESH)
        pl.semaphore_wait(barrier, 2)

        for k in range(YAHEAD):
            issue_ysend(k)

        for k in range(NCH):
            s = k % NSX
            if k + LAHEAD < NCH:
                loads[k + LAHEAD] = start_loads(k + LAHEAD)
            if k + YAHEAD < NCH:
                issue_ysend(k + YAHEAD)

            ys[k].wait_recv()
            loads[k][1].wait()
            if k >= NSX:
                sum_stores[k - NSX].wait()
                xs[k - NSX].wait_send()
            xsend[s] = (
                b_f32[k % NS] + yrecv[pl.ds(k * RCH, RCH), :].astype(
                    jnp.float32)
            ).astype(jnp.bfloat16)

            st = pltpu.make_async_copy(
                xsend.at[s], out_hbm.at[pl.ds(row0 + k * RCH, RCH), :],
                sum_store_sems.at[s])
            st.start()
            sum_stores[k] = st
            xs[k] = x_rdma(k)
            xs[k].start()

            if k >= 1:
                drain_xrecv(k - 1)

        drain_xrecv(NCH - 1)
        for k in range(NCH - NSX, NCH):
            sum_stores[k].wait()
            xs[k].wait_send()
        for k in range(NCH):
            if k not in ys_waited:
                ys[k].wait_send()
        for k in range(NCH - NSX, NCH):
            xstores[k].wait()

    return pl.pallas_call(
        body,
        out_shape=jax.ShapeDtypeStruct((M, NOUT), jnp.bfloat16),
        in_specs=[pl.BlockSpec(memory_space=pl.ANY)],
        out_specs=pl.BlockSpec(memory_space=pltpu.MemorySpace.HBM),
        scratch_shapes=[
            pltpu.VMEM((NS, RCH, NOUT), jnp.float32),
            pltpu.VMEM((NS, RCH, NOUT), jnp.float32),
            pltpu.VMEM((NS, RCH, NOUT), jnp.bfloat16),
            pltpu.VMEM((HALF, NOUT), jnp.bfloat16),
            pltpu.VMEM((NSX, RCH, NOUT), jnp.bfloat16),
            pltpu.VMEM((HALF, NOUT), jnp.bfloat16),
            pltpu.SemaphoreType.DMA((NS,)),
            pltpu.SemaphoreType.DMA((NS,)),
            pltpu.SemaphoreType.DMA((NSX,)),
            pltpu.SemaphoreType.DMA((NSX,)),
            pltpu.SemaphoreType.DMA((NS,)),
            pltpu.SemaphoreType.DMA((NCH,)),
            pltpu.SemaphoreType.DMA((NSX,)),
            pltpu.SemaphoreType.DMA((NCH,)),
        ],
        compiler_params=pltpu.CompilerParams(
            collective_id=0, vmem_limit_bytes=100 * 1024 * 1024),
    )(x)
